# baseline (speedup 1.0000x reference)
import functools

import jax
import jax.numpy as jnp
from jax import lax
from jax.experimental import pallas as pl
from jax.experimental.pallas import tpu as pltpu
from jax.experimental.pallas import tpu_sc as plsc

N = 50000
E = 1600000
D = 3
HD = 64
H_SMOOTH = 0.5

NW = 32
TD = 16
SLAB_ROWS = 800
MACRO = 6400
SUBS = MACRO // SLAB_ROWS
NMACRO = 8
EPW = MACRO * NMACRO
EPAD = NW * EPW
NSTEP = NMACRO * SUBS

EDGE_BLK = 3200

NNH = N // 2
NNP = 25600
STRIPE = NNP // 16
CHN = 2048
EPW2 = EPAD // 16
CPL = EPW2 // CHN


def _sc_gather_body(v4_hbm, i2_hbm, j2_hbm, outi_hbm, outj_hbm,
                    idxIA, idxIB, idxJA, idxJB,
                    slabIA, slabIB, slabJA, slabJB,
                    sem_g0, sem_g1, sem_o0, sem_o1):
    core = lax.axis_index("c")
    sub = lax.axis_index("s")
    wid = sub * 2 + core
    base_rows = wid * EPW

    idxI = (idxIA, idxIB)
    idxJ = (idxJA, idxJB)
    slabsI = (slabIA, slabIB)
    slabsJ = (slabJA, slabJB)
    sems_g = (sem_g0, sem_g1)
    sems_o = (sem_o0, sem_o1)

    def wait_gathers(par):
        pltpu.make_async_copy(
            v4_hbm.at[idxI[0].at[pl.ds(0, SLAB_ROWS)]], slabsI[par],
            sems_g[par]).wait()
        pltpu.make_async_copy(
            v4_hbm.at[idxJ[0].at[pl.ds(0, SLAB_ROWS)]], slabsJ[par],
            sems_g[par]).wait()

    def fire_store(s, par):
        row0 = base_rows + s * SLAB_ROWS
        pltpu.async_copy(slabsI[par],
                         outi_hbm.at[pl.ds(row0, SLAB_ROWS)], sems_o[par])
        pltpu.async_copy(slabsJ[par],
                         outj_hbm.at[pl.ds(row0, SLAB_ROWS)], sems_o[par])

    def drain_store(par):
        pltpu.make_async_copy(
            slabsI[par], outi_hbm.at[pl.ds(0, SLAB_ROWS)], sems_o[par]).wait()
        pltpu.make_async_copy(
            slabsJ[par], outj_hbm.at[pl.ds(0, SLAB_ROWS)], sems_o[par]).wait()

    def macro_pair(m2, carry):
        for mh in range(2):
            m = m2 * 2 + mh
            pltpu.sync_copy(i2_hbm.at[wid, pl.ds(m * MACRO, MACRO)], idxI[mh])
            pltpu.sync_copy(j2_hbm.at[wid, pl.ds(m * MACRO, MACRO)], idxJ[mh])

            def sub2(t2, carry2, _m=m, _mh=mh):
                for half in range(2):
                    t = t2 * 2 + half
                    s = _m * SUBS + t
                    par = half

                    @pl.when(s >= 2)
                    def _():
                        drain_store(par)
                    pltpu.async_copy(
                        v4_hbm.at[idxI[_mh].at[pl.ds(t * SLAB_ROWS,
                                                     SLAB_ROWS)]],
                        slabsI[par], sems_g[par])
                    pltpu.async_copy(
                        v4_hbm.at[idxJ[_mh].at[pl.ds(t * SLAB_ROWS,
                                                     SLAB_ROWS)]],
                        slabsJ[par], sems_g[par])

                    @pl.when(s >= 1)
                    def _():
                        wait_gathers(1 - par)
                        fire_store(s - 1, 1 - par)
                return carry2

            lax.fori_loop(0, SUBS // 2, sub2, 0)
        return carry

    lax.fori_loop(0, NMACRO // 2, macro_pair, 0)

    last_par = (NSTEP - 1) % 2
    wait_gathers(last_par)
    fire_store(NSTEP - 1, last_par)
    drain_store(0)
    drain_store(1)


def _sc_gather(v4, i2, j2):
    mesh = plsc.VectorSubcoreMesh(core_axis_name="c", subcore_axis_name="s")
    f = pl.kernel(
        _sc_gather_body,
        out_type=[
            jax.ShapeDtypeStruct((EPAD, TD), jnp.float32),
            jax.ShapeDtypeStruct((EPAD, TD), jnp.float32),
        ],
        mesh=mesh,
        scratch_types=[
            pltpu.VMEM((MACRO,), jnp.int32),
            pltpu.VMEM((MACRO,), jnp.int32),
            pltpu.VMEM((MACRO,), jnp.int32),
            pltpu.VMEM((MACRO,), jnp.int32),
            pltpu.VMEM((SLAB_ROWS, TD), jnp.float32),
            pltpu.VMEM((SLAB_ROWS, TD), jnp.float32),
            pltpu.VMEM((SLAB_ROWS, TD), jnp.float32),
            pltpu.VMEM((SLAB_ROWS, TD), jnp.float32),
            pltpu.SemaphoreType.DMA,
            pltpu.SemaphoreType.DMA,
            pltpu.SemaphoreType.DMA,
            pltpu.SemaphoreType.DMA,
        ],
        compiler_params=pltpu.CompilerParams(use_tc_tiling_on_sc=False),
    )
    return f(v4, i2, j2)


def _sc_scatter_body(ti_hbm, tj_hbm, in_hbm, jn_hbm, out_hbm,
                     idxI0, idxI1, valI0, valI1, idxJ0, idxJ1, valJ0, valJ1,
                     ones, zbuf, bsi, bci, bsj, bcj, bres,
                     sh_si, sh_ci, sh_sj, sh_cj,
                     sem_l0, sem_l1, sem_s):
    core = lax.axis_index("c")
    sub = lax.axis_index("s")
    nodebase = core * NNH
    ebase = sub * EPW2

    idxI = (idxI0, idxI1)
    valI = (valI0, valI1)
    idxJ = (idxJ0, idxJ1)
    valJ = (valJ0, valJ1)
    sems_l = (sem_l0, sem_l1)
    shareds = (sh_si, sh_ci, sh_sj, sh_cj)

    def fill(k, carry):
        ones[pl.ds(k * 16, 16)] = jnp.full((16,), 1.0, jnp.float32)
        return carry
    lax.fori_loop(0, CHN // 16, fill, 0)

    def zfill(k, carry):
        zbuf[pl.ds(k * 16, 16)] = jnp.zeros((16,), jnp.float32)
        return carry
    lax.fori_loop(0, STRIPE // 16, zfill, 0)
    for arr in shareds:
        pltpu.sync_copy(zbuf, arr.at[pl.ds(sub * STRIPE, STRIPE)])
    plsc.subcore_barrier()

    def fire_loads(c, p):
        off = ebase + c * CHN
        pltpu.async_copy(in_hbm.at[pl.ds(off, CHN)], idxI[p], sems_l[p])
        pltpu.async_copy(ti_hbm.at[pl.ds(off, CHN)], valI[p], sems_l[p])
        pltpu.async_copy(jn_hbm.at[pl.ds(off, CHN)], idxJ[p], sems_l[p])
        pltpu.async_copy(tj_hbm.at[pl.ds(off, CHN)], valJ[p], sems_l[p])

    def drain_loads(p):
        pltpu.make_async_copy(in_hbm.at[pl.ds(0, CHN)], idxI[p],
                              sems_l[p]).wait()
        pltpu.make_async_copy(ti_hbm.at[pl.ds(0, CHN)], valI[p],
                              sems_l[p]).wait()
        pltpu.make_async_copy(jn_hbm.at[pl.ds(0, CHN)], idxJ[p],
                              sems_l[p]).wait()
        pltpu.make_async_copy(tj_hbm.at[pl.ds(0, CHN)], valJ[p],
                              sems_l[p]).wait()

    def remap(p):
        def body(k, carry):
            sl = pl.ds(k * 16, 16)
            for buf in (idxI[p], idxJ[p]):
                x = buf[sl]
                loc = x - nodebase
                ok = (loc >= 0) & (loc < NNH)
                buf[sl] = jnp.where(ok, loc, NNH)
            return carry
        lax.fori_loop(0, CHN // 16, body, 0)

    def fire_scatters(p):
        pltpu.async_copy(valI[p], sh_si.at[idxI[p]], sem_s, add=True)
        pltpu.async_copy(ones, sh_ci.at[idxI[p]], sem_s, add=True)
        pltpu.async_copy(valJ[p], sh_sj.at[idxJ[p]], sem_s, add=True)
        pltpu.async_copy(ones, sh_cj.at[idxJ[p]], sem_s, add=True)

    def drain_scatters(p):
        pltpu.make_async_copy(valI[p], sh_si.at[idxI[p]], sem_s).wait()
        pltpu.make_async_copy(ones, sh_ci.at[idxI[p]], sem_s).wait()
        pltpu.make_async_copy(valJ[p], sh_sj.at[idxJ[p]], sem_s).wait()
        pltpu.make_async_copy(ones, sh_cj.at[idxJ[p]], sem_s).wait()

    fire_loads(0, 0)

    def chunk_pair(c2, carry):
        for p in range(2):
            c = c2 * 2 + p
            drain_loads(p)
            remap(p)

            @pl.when(c >= 1)
            def _():
                drain_scatters(1 - p)
            fire_scatters(p)

            @pl.when(c < CPL - 1)
            def _():
                fire_loads(c + 1, 1 - p)
        return carry

    lax.fori_loop(0, CPL // 2, chunk_pair, 0)
    drain_scatters((CPL - 1) % 2)
    plsc.subcore_barrier()

    s0 = sub * STRIPE
    pltpu.sync_copy(sh_si.at[pl.ds(s0, STRIPE)], bsi)
    pltpu.sync_copy(sh_ci.at[pl.ds(s0, STRIPE)], bci)
    pltpu.sync_copy(sh_sj.at[pl.ds(s0, STRIPE)], bsj)
    pltpu.sync_copy(sh_cj.at[pl.ds(s0, STRIPE)], bcj)

    def div(k, carry):
        sl = pl.ds(k * 16, 16)
        res = (bsi[sl] / jnp.maximum(bci[sl], 1.0)
               + bsj[sl] / jnp.maximum(bcj[sl], 1.0))
        bres[sl] = res
        return carry
    lax.fori_loop(0, STRIPE // 16, div, 0)

    node0 = nodebase + s0
    nvalid = NNH - 15 * STRIPE

    @pl.when(sub < 15)
    def _():
        pltpu.sync_copy(bres, out_hbm.at[pl.ds(node0, STRIPE)])

    @pl.when(sub == 15)
    def _():
        pltpu.sync_copy(bres.at[pl.ds(0, nvalid)],
                        out_hbm.at[pl.ds(node0, nvalid)])


def _sc_scatter(ti_pad, tj_pad, i_n, j_n):
    mesh = plsc.VectorSubcoreMesh(core_axis_name="c", subcore_axis_name="s")
    f = pl.kernel(
        _sc_scatter_body,
        out_type=jax.ShapeDtypeStruct((N,), jnp.float32),
        mesh=mesh,
        scratch_types=(
            [pltpu.VMEM((CHN,), jnp.int32) for _ in range(2)]
            + [pltpu.VMEM((CHN,), jnp.float32) for _ in range(2)]
            + [pltpu.VMEM((CHN,), jnp.int32) for _ in range(2)]
            + [pltpu.VMEM((CHN,), jnp.float32) for _ in range(2)]
            + [pltpu.VMEM((CHN,), jnp.float32)]
            + [pltpu.VMEM((STRIPE,), jnp.float32) for _ in range(6)]
            + [pltpu.VMEM_SHARED((NNP,), jnp.float32) for _ in range(4)]
            + [pltpu.SemaphoreType.DMA for _ in range(3)]
        ),
        compiler_params=pltpu.CompilerParams(use_tc_tiling_on_sc=False),
    )
    return f(ti_pad, tj_pad, i_n, j_n)


def _silu(x):
    return x * jax.nn.sigmoid(x)


def _mlp_body(r_ref, vi_ref, vj_ref, W1_ref, b1_ref, W2_ref, b2_ref, W3_ref,
              b3_ref, ti_ref, tj_ref):
    r_blk = r_ref[...]
    vij = vi_ref[:, 0:4] - vj_ref[:, 0:4]
    W1 = W1_ref[...]
    b1 = b1_ref[...]
    W2 = W2_ref[...]
    b2 = b2_ref[...]
    W3 = W3_ref[...]
    b3 = b3_ref[0, 0]

    r = jnp.sqrt(jnp.sum(r_blk * r_blk, axis=1, keepdims=True))
    rn = r * (1.0 / H_SMOOTH)
    w_r = W1[:, 0:1].T
    W_v = W1[:, 1:4]
    a = rn * w_r + b1
    c = lax.dot_general(vij[:, 0:3], W_v, (((1,), (1,)), ((), ())),
                        preferred_element_type=jnp.float32)
    h_i = _silu(a + c)
    h_j = _silu(a - c)
    z_i = _silu(lax.dot_general(h_i, W2, (((1,), (1,)), ((), ())),
                                preferred_element_type=jnp.float32) + b2)
    z_j = _silu(lax.dot_general(h_j, W2, (((1,), (1,)), ((), ())),
                                preferred_element_type=jnp.float32) + b2)
    ti_ref[...] = jnp.sum(z_i * W3, axis=1, keepdims=True) + b3
    tj_ref[...] = jnp.sum(z_j * W3, axis=1, keepdims=True) + b3


def _edge_mlp(r_ij, vi_pad, vj_pad, W1, b1, W2, b2, W3, b3):
    nblk = E // EDGE_BLK
    blk = lambda i: (i, 0)
    full = lambda i: (0, 0)
    return pl.pallas_call(
        _mlp_body,
        grid=(nblk,),
        in_specs=[
            pl.BlockSpec((EDGE_BLK, 3), blk),
            pl.BlockSpec((EDGE_BLK, TD), blk),
            pl.BlockSpec((EDGE_BLK, TD), blk),
            pl.BlockSpec((HD, 4), full),
            pl.BlockSpec((1, HD), full),
            pl.BlockSpec((HD, HD), full),
            pl.BlockSpec((1, HD), full),
            pl.BlockSpec((1, HD), full),
            pl.BlockSpec((1, 1), full),
        ],
        out_specs=[
            pl.BlockSpec((EDGE_BLK, 1), blk),
            pl.BlockSpec((EDGE_BLK, 1), blk),
        ],
        out_shape=[
            jax.ShapeDtypeStruct((E, 1), jnp.float32),
            jax.ShapeDtypeStruct((E, 1), jnp.float32),
        ],
    )(r_ij, vi_pad, vj_pad, W1, b1.reshape(1, HD), W2, b2.reshape(1, HD), W3,
      b3.reshape(1, 1))


def kernel(edge_index, r_ij, v, W1, b1, W2, b2, W3, b3):
    i = edge_index[0]
    j = edge_index[1]
    v4 = jnp.pad(v, ((0, 0), (0, TD - D)))
    i2 = jnp.pad(i, (0, EPAD - E)).reshape(NW, EPW)
    j2 = jnp.pad(j, (0, EPAD - E)).reshape(NW, EPW)
    vi_pad, vj_pad = _sc_gather(v4, i2, j2)
    t_i, t_j = _edge_mlp(r_ij, vi_pad, vj_pad, W1, b1, W2, b2, W3, b3)
    ti_pad = jnp.pad(t_i[:, 0], (0, EPAD - E))
    tj_pad = jnp.pad(t_j[:, 0], (0, EPAD - E))
    i_n = jnp.pad(i, (0, EPAD - E), constant_values=N)
    j_n = jnp.pad(j, (0, EPAD - E), constant_values=N)
    return _sc_scatter(ti_pad, tj_pad, i_n, j_n).reshape(N, 1)
